# baseline (device time: 208209 ns/iter reference)
import jax
import jax.numpy as jnp
from jax import lax
from jax.experimental import pallas as pl
from jax.experimental.pallas import tpu as pltpu

N_DEV = 16
S_LOC = 256
BH = 16
D = 64
SCALE = D ** -0.5


def _body(q_ref, kv_ref, o_ref, kv_all, acc_ref, m_ref, l_ref,
          z_send, z_recv, cw_send, cw_recv, ccw_send, ccw_recv):
    my = lax.axis_index("i")
    qpos = lax.rem(my, 4)
    base = my - qpos
    nxt = base + lax.rem(qpos + 1, 4)
    prv = base + lax.rem(qpos + 3, 4)
    up4 = lax.rem(my + 4, N_DEV)
    dn4 = lax.rem(my + 12, N_DEV)

    barrier_sem = pltpu.get_barrier_semaphore()
    for nbr in (nxt, prv, up4, dn4):
        pl.semaphore_signal(
            barrier_sem, inc=1,
            device_id=(nbr,), device_id_type=pl.DeviceIdType.MESH,
        )
    pl.semaphore_wait(barrier_sem, 4)

    HB = BH // 2

    def rdma(src_slot, dst_slot, tgt, send_sems, recv_sems, idx, part):
        src = kv_ref.at[...] if src_slot == 0 else kv_all.at[src_slot]
        lo = part * HB
        return pltpu.make_async_remote_copy(
            src_ref=src.at[:, lo:lo + HB],
            dst_ref=kv_all.at[dst_slot].at[:, lo:lo + HB],
            send_sem=send_sems.at[idx, part],
            recv_sem=recv_sems.at[idx, part],
            device_id=(tgt,),
            device_id_type=pl.DeviceIdType.MESH,
        )

    def flow(src_slot, dst_slot, tgt, send_sems, recv_sems, idx):
        return [
            rdma(src_slot, dst_slot, tgt, send_sems, recv_sems, idx, p)
            for p in (0, 1)
        ]

    U0 = flow(0, 12, up4, z_send, z_recv, 0)
    U1 = flow(12, 8, up4, z_send, z_recv, 1)
    D0 = flow(0, 4, dn4, z_send, z_recv, 2)
    cw = [
        flow(0, 3, nxt, cw_send, cw_recv, 0),
        flow(4, 7, nxt, cw_send, cw_recv, 1),
        flow(8, 11, nxt, cw_send, cw_recv, 2),
        flow(12, 15, nxt, cw_send, cw_recv, 3),
        flow(3, 2, nxt, cw_send, cw_recv, 4),
        flow(7, 6, nxt, cw_send, cw_recv, 5),
    ]
    ccw = [
        flow(0, 1, prv, ccw_send, ccw_recv, 0),
        flow(4, 5, prv, ccw_send, ccw_recv, 1),
        flow(8, 9, prv, ccw_send, ccw_recv, 2),
        flow(12, 13, prv, ccw_send, ccw_recv, 3),
        flow(9, 10, prv, ccw_send, ccw_recv, 4),
        flow(13, 14, prv, ccw_send, ccw_recv, 5),
    ]

    def arrive(src_flow, then=()):
        for p in (0, 1):
            src_flow[p].wait_recv()
            for f in then:
                f[p].start()

    def process(slot, first=False):
        def bh_step(bh, carry):
            q = q_ref[bh]
            k = kv_all[slot, 0, bh]
            v = kv_all[slot, 1, bh]
            s = lax.dot_general(
                q, k, (((1,), (1,)), ((), ())),
                preferred_element_type=jnp.float32,
            )
            m_blk = jnp.max(s, axis=-1, keepdims=True)
            if first:
                m_new = m_blk
                p = jnp.exp(s - m_new)
                l_ref[bh] = jnp.sum(p, axis=-1, keepdims=True)
                acc_ref[bh] = jnp.dot(
                    p.astype(jnp.bfloat16), v,
                    preferred_element_type=jnp.float32,
                )
            else:
                m_old = m_ref[bh]
                m_new = jnp.maximum(m_old, m_blk)
                alpha = jnp.exp(m_old - m_new)
                p = jnp.exp(s - m_new)
                l_ref[bh] = l_ref[bh] * alpha + jnp.sum(
                    p, axis=-1, keepdims=True
                )
                acc_ref[bh] = acc_ref[bh] * alpha + jnp.dot(
                    p.astype(jnp.bfloat16), v,
                    preferred_element_type=jnp.float32,
                )
            m_ref[bh] = m_new
            return carry

        lax.fori_loop(0, BH, bh_step, 0)

    for p in (0, 1):
        U0[p].start()
        D0[p].start()
        cw[0][p].start()
        ccw[0][p].start()
    kv_all[0] = kv_ref[...]
    process(0, first=True)

    arrive(U0, then=(U1, cw[3], ccw[3]))
    process(12)
    arrive(D0, then=(cw[1], ccw[1]))
    process(4)
    arrive(U1, then=(cw[2], ccw[2]))
    process(8)

    arrive(cw[0], then=(cw[4],))
    process(3)
    arrive(ccw[0])
    process(1)
    arrive(cw[1], then=(cw[5],))
    process(7)
    arrive(ccw[1])
    process(5)
    arrive(cw[2])
    process(11)
    arrive(ccw[2], then=(ccw[4],))
    process(9)
    arrive(cw[3])
    process(15)
    arrive(ccw[3], then=(ccw[5],))
    process(13)
    arrive(cw[4])
    process(2)
    arrive(cw[5])
    process(6)
    arrive(ccw[4])
    process(10)
    arrive(ccw[5])
    process(14)

    for bh in range(BH):
        o_ref[bh] = acc_ref[bh] / l_ref[bh]

    for fl in [U0, U1, D0] + cw + ccw:
        for r in fl:
            r.wait_send()


def kernel(Q, K, V):
    qt = (jnp.transpose(Q, (0, 2, 1, 3)).reshape(BH, S_LOC, D) * SCALE).astype(
        jnp.bfloat16
    )
    kt = jnp.transpose(K, (0, 2, 1, 3)).reshape(BH, S_LOC, D).astype(jnp.bfloat16)
    vt = jnp.transpose(V, (0, 2, 1, 3)).reshape(BH, S_LOC, D).astype(jnp.bfloat16)
    kv = jnp.stack([kt, vt])

    out = pl.pallas_call(
        _body,
        out_shape=jax.ShapeDtypeStruct((BH, S_LOC, D), jnp.float32),
        in_specs=[
            pl.BlockSpec(memory_space=pltpu.VMEM),
            pl.BlockSpec(memory_space=pltpu.VMEM),
        ],
        out_specs=pl.BlockSpec(memory_space=pltpu.VMEM),
        scratch_shapes=[
            pltpu.VMEM((N_DEV, 2, BH, S_LOC, D), jnp.bfloat16),
            pltpu.VMEM((BH, S_LOC, D), jnp.float32),
            pltpu.VMEM((BH, S_LOC, 1), jnp.float32),
            pltpu.VMEM((BH, S_LOC, 1), jnp.float32),
            pltpu.SemaphoreType.DMA((3, 2)),
            pltpu.SemaphoreType.DMA((3, 2)),
            pltpu.SemaphoreType.DMA((6, 2)),
            pltpu.SemaphoreType.DMA((6, 2)),
            pltpu.SemaphoreType.DMA((6, 2)),
            pltpu.SemaphoreType.DMA((6, 2)),
        ],
        compiler_params=pltpu.CompilerParams(
            collective_id=0, vmem_limit_bytes=100 * 1024 * 1024
        ),
    )(qt, kv)

    return jnp.transpose(out.reshape(2, 8, S_LOC, D), (0, 2, 1, 3))
